# speedup vs baseline: 1.2524x; 1.2524x over previous
import functools

import jax
import jax.numpy as jnp
from jax.experimental import pallas as pl
from jax.experimental.pallas import tpu as pltpu

SEQ, B = 4096, 4
NUM_SEGMENTS = 8
EMB_DIM = 128
OUT_DIM = 1024
N_TOKENS = SEQ * B

BLOCK = 512
NUM_BLOCKS = N_TOKENS // BLOCK


def _tc_kernel(seg_ref, table_ref, w_ref, out_ref, p_ref):
    @pl.when(pl.program_id(0) == 0)
    def _():
        p_ref[...] = jax.lax.dot_general(
            table_ref[...], w_ref[...],
            dimension_numbers=(((1,), (1,)), ((), ())),
            preferred_element_type=jnp.float32,
        )

    seg = seg_ref[0, 0, :]
    onehot = (seg[:, None] == jax.lax.broadcasted_iota(jnp.int32, (BLOCK, NUM_SEGMENTS), 1)).astype(jnp.float32)
    out_ref[...] = jax.lax.dot_general(
        onehot, p_ref[...],
        dimension_numbers=(((1,), (0,)), ((), ())),
        preferred_element_type=jnp.float32,
    )


@jax.jit
def kernel(input, align_pos, segment_ids, table, W):
    seg = segment_ids.astype(jnp.int32).reshape(NUM_BLOCKS, 1, BLOCK)
    out = pl.pallas_call(
        _tc_kernel,
        grid=(NUM_BLOCKS,),
        in_specs=[
            pl.BlockSpec((1, 1, BLOCK), lambda i: (i, 0, 0)),
            pl.BlockSpec((NUM_SEGMENTS, EMB_DIM), lambda i: (0, 0)),
            pl.BlockSpec((OUT_DIM, EMB_DIM), lambda i: (0, 0)),
        ],
        out_specs=pl.BlockSpec((BLOCK, OUT_DIM), lambda i: (i, 0)),
        out_shape=jax.ShapeDtypeStruct((N_TOKENS, OUT_DIM), jnp.float32),
        scratch_shapes=[pltpu.VMEM((NUM_SEGMENTS, OUT_DIM), jnp.float32)],
    )(seg, table, W)
    return out.reshape(SEQ, B, OUT_DIM)
